# baseline (device time: 55157 ns/iter reference)
import jax
import jax.numpy as jnp
from jax import lax
from jax.experimental import pallas as pl
from jax.experimental.pallas import tpu as pltpu

N_DEV = 4
N_GLOBAL = 8192
EPS = 1e-5
BLK = 512
COMM = True


def kernel(x, gamma):
    m, n_local = x.shape
    n_blocks = m // BLK
    gamma2d = gamma.reshape(1, n_local)

    def body(x_hbm, gamma_ref, out_ref, xbuf, p_col, p_row, comm_row,
             copy_sems, send_sems, recv_sems):
        my = lax.axis_index("i")

        if COMM:
            barrier_sem = pltpu.get_barrier_semaphore()
            for o in range(1, N_DEV):
                pl.semaphore_signal(
                    barrier_sem, inc=1,
                    device_id=(lax.rem(my + o, N_DEV),),
                    device_id_type=pl.DeviceIdType.MESH,
                )
            pl.semaphore_wait(barrier_sem, N_DEV - 1)

        def in_copy(i, slot):
            return pltpu.make_async_copy(
                x_hbm.at[pl.ds(i * BLK, BLK), :],
                xbuf.at[slot],
                copy_sems.at[slot],
            )

        in_copy(0, 0).start()
        for i in range(n_blocks):
            slot = i % 2
            if i + 1 < n_blocks:
                in_copy(i + 1, (i + 1) % 2).start()
            in_copy(i, slot).wait()
            xb = xbuf[slot]
            p_col[pl.ds(i * BLK, BLK), :] = jnp.sum(
                xb * xb, axis=1, keepdims=True)
            out_ref[pl.ds(i * BLK, BLK), :] = xb.astype(jnp.bfloat16)

        e11 = jnp.ones((1, 1), jnp.float32)
        p_row[:, :] = lax.dot_general(
            e11, p_col[:, :], (((1,), (1,)), ((), ())),
            preferred_element_type=jnp.float32)

        if COMM:
            rdmas = []
            for o in range(1, N_DEV):
                rdma = pltpu.make_async_remote_copy(
                    src_ref=p_row,
                    dst_ref=comm_row.at[o - 1],
                    send_sem=send_sems.at[o - 1],
                    recv_sem=recv_sems.at[o - 1],
                    device_id=(lax.rem(my + o, N_DEV),),
                    device_id_type=pl.DeviceIdType.MESH,
                )
                rdma.start()
                rdmas.append(rdma)
            for rdma in rdmas:
                rdma.wait()

            total = (p_row[:, :] + comm_row[0] + comm_row[1] + comm_row[2])
        else:
            total = p_row[:, :] * 4.0

        inv_row = lax.rsqrt(total * (1.0 / N_GLOBAL) + EPS)
        p_col[:, :] = lax.dot_general(
            inv_row, e11, (((0,), (0,)), ((), ())),
            preferred_element_type=jnp.float32)

        g = gamma_ref[:, :]
        for i in range(n_blocks):
            sl = pl.ds(i * BLK, BLK)
            xb = out_ref[sl, :].astype(jnp.float32)
            inv = p_col[sl, :]
            out_ref[sl, :] = (g * xb * inv).astype(jnp.bfloat16)

    return pl.pallas_call(
        body,
        out_shape=jax.ShapeDtypeStruct((m, n_local), jnp.bfloat16),
        in_specs=[
            pl.BlockSpec(memory_space=pl.ANY),
            pl.BlockSpec(memory_space=pltpu.VMEM),
        ],
        out_specs=pl.BlockSpec(memory_space=pltpu.VMEM),
        scratch_shapes=[
            pltpu.VMEM((2, BLK, n_local), jnp.float32),
            pltpu.VMEM((m, 1), jnp.float32),
            pltpu.VMEM((1, m), jnp.float32),
            pltpu.VMEM((N_DEV - 1, 1, m), jnp.float32),
            pltpu.SemaphoreType.DMA((2,)),
            pltpu.SemaphoreType.DMA((N_DEV - 1,)),
            pltpu.SemaphoreType.DMA((N_DEV - 1,)),
        ],
        compiler_params=pltpu.CompilerParams(
            collective_id=0 if COMM else None,
            vmem_limit_bytes=60 * 1024 * 1024,
        ),
    )(x, gamma2d)


# device time: 53620 ns/iter; 1.0287x vs baseline; 1.0287x over previous
import jax
import jax.numpy as jnp
from jax import lax
from jax.experimental import pallas as pl
from jax.experimental.pallas import tpu as pltpu

N_DEV = 4
N_GLOBAL = 8192
EPS = 1e-5
BLK = 512
CHUNK_BLKS = 4
COMM = True


def kernel(x, gamma):
    m, n_local = x.shape
    n_blocks = m // BLK
    gamma2d = gamma.reshape(1, n_local)

    def body(x_hbm, gamma_ref, out_ref, xbuf, p_col, p_row, comm_row,
             copy_sems, send_sems, recv_sems):
        my = lax.axis_index("i")

        if COMM:
            barrier_sem = pltpu.get_barrier_semaphore()
            for o in range(1, N_DEV):
                pl.semaphore_signal(
                    barrier_sem, inc=1,
                    device_id=(lax.rem(my + o, N_DEV),),
                    device_id_type=pl.DeviceIdType.MESH,
                )
            pl.semaphore_wait(barrier_sem, N_DEV - 1)

        def in_copy(i, slot):
            return pltpu.make_async_copy(
                x_hbm.at[pl.ds(i * BLK, BLK), :],
                xbuf.at[slot],
                copy_sems.at[slot],
            )

        e11 = jnp.ones((1, 1), jnp.float32)
        crows = CHUNK_BLKS * BLK
        n_chunks = n_blocks // CHUNK_BLKS
        rdmas = []

        def send_chunk(c):
            csl = pl.ds(c * crows, crows)
            p_row[:, csl] = lax.dot_general(
                e11, p_col[csl, :], (((1,), (1,)), ((), ())),
                preferred_element_type=jnp.float32)
            if not COMM:
                return
            hs = []
            for o in range(1, N_DEV):
                rdma = pltpu.make_async_remote_copy(
                    src_ref=p_row.at[:, csl],
                    dst_ref=comm_row.at[o - 1, :, csl],
                    send_sem=send_sems.at[o - 1, c],
                    recv_sem=recv_sems.at[o - 1, c],
                    device_id=(lax.rem(my + o, N_DEV),),
                    device_id_type=pl.DeviceIdType.MESH,
                )
                rdma.start()
                hs.append(rdma)
            rdmas.append(hs)

        in_copy(0, 0).start()
        for i in range(n_blocks):
            slot = i % 2
            if i + 1 < n_blocks:
                in_copy(i + 1, (i + 1) % 2).start()
            in_copy(i, slot).wait()
            xb = xbuf[slot]
            p_col[pl.ds(i * BLK, BLK), :] = jnp.sum(
                xb * xb, axis=1, keepdims=True)
            out_ref[pl.ds(i * BLK, BLK), :] = xb.astype(jnp.bfloat16)
            if (i + 1) % CHUNK_BLKS == 0:
                send_chunk(i // CHUNK_BLKS)

        g = gamma_ref[:, :]
        for c in range(n_chunks):
            csl = pl.ds(c * crows, crows)
            if COMM:
                for rdma in rdmas[c]:
                    rdma.wait()
                total = (p_row[:, csl] + comm_row[0, :, csl] +
                         comm_row[1, :, csl] + comm_row[2, :, csl])
            else:
                total = p_row[:, csl] * 4.0
            inv_row = lax.rsqrt(total * (1.0 / N_GLOBAL) + EPS)
            p_col[csl, :] = lax.dot_general(
                inv_row, e11, (((0,), (0,)), ((), ())),
                preferred_element_type=jnp.float32)
            for i in range(c * CHUNK_BLKS, (c + 1) * CHUNK_BLKS):
                sl = pl.ds(i * BLK, BLK)
                xb = out_ref[sl, :].astype(jnp.float32)
                inv = p_col[sl, :]
                out_ref[sl, :] = (g * xb * inv).astype(jnp.bfloat16)

    return pl.pallas_call(
        body,
        out_shape=jax.ShapeDtypeStruct((m, n_local), jnp.bfloat16),
        in_specs=[
            pl.BlockSpec(memory_space=pl.ANY),
            pl.BlockSpec(memory_space=pltpu.VMEM),
        ],
        out_specs=pl.BlockSpec(memory_space=pltpu.VMEM),
        scratch_shapes=[
            pltpu.VMEM((2, BLK, n_local), jnp.float32),
            pltpu.VMEM((m, 1), jnp.float32),
            pltpu.VMEM((1, m), jnp.float32),
            pltpu.VMEM((N_DEV - 1, 1, m), jnp.float32),
            pltpu.SemaphoreType.DMA((2,)),
            pltpu.SemaphoreType.DMA(
                (N_DEV - 1, m // (CHUNK_BLKS * BLK))),
            pltpu.SemaphoreType.DMA(
                (N_DEV - 1, m // (CHUNK_BLKS * BLK))),
        ],
        compiler_params=pltpu.CompilerParams(
            collective_id=0 if COMM else None,
            vmem_limit_bytes=60 * 1024 * 1024,
        ),
    )(x, gamma2d)


# device time: 53395 ns/iter; 1.0330x vs baseline; 1.0042x over previous
import jax
import jax.numpy as jnp
from jax import lax
from jax.experimental import pallas as pl
from jax.experimental.pallas import tpu as pltpu

N_DEV = 4
N_GLOBAL = 8192
EPS = 1e-5
BLK = 512
BARRIER = True
CHUNK_BLKS = 4
COMM = False


def kernel(x, gamma):
    m, n_local = x.shape
    n_blocks = m // BLK
    gamma2d = gamma.reshape(1, n_local)

    def body(x_hbm, gamma_ref, out_ref, xbuf, p_col, p_row, comm_row,
             copy_sems, send_sems, recv_sems):
        my = lax.axis_index("i")

        if COMM or BARRIER:
            barrier_sem = pltpu.get_barrier_semaphore()
            for o in range(1, N_DEV):
                pl.semaphore_signal(
                    barrier_sem, inc=1,
                    device_id=(lax.rem(my + o, N_DEV),),
                    device_id_type=pl.DeviceIdType.MESH,
                )
            pl.semaphore_wait(barrier_sem, N_DEV - 1)

        def in_copy(i, slot):
            return pltpu.make_async_copy(
                x_hbm.at[pl.ds(i * BLK, BLK), :],
                xbuf.at[slot],
                copy_sems.at[slot],
            )

        e11 = jnp.ones((1, 1), jnp.float32)
        crows = CHUNK_BLKS * BLK
        n_chunks = n_blocks // CHUNK_BLKS
        rdmas = []

        def send_chunk(c):
            csl = pl.ds(c * crows, crows)
            p_row[:, csl] = lax.dot_general(
                e11, p_col[csl, :], (((1,), (1,)), ((), ())),
                preferred_element_type=jnp.float32)
            if not COMM:
                return
            hs = []
            for o in range(1, N_DEV):
                rdma = pltpu.make_async_remote_copy(
                    src_ref=p_row.at[:, csl],
                    dst_ref=comm_row.at[o - 1, :, csl],
                    send_sem=send_sems.at[o - 1, c],
                    recv_sem=recv_sems.at[o - 1, c],
                    device_id=(lax.rem(my + o, N_DEV),),
                    device_id_type=pl.DeviceIdType.MESH,
                )
                rdma.start()
                hs.append(rdma)
            rdmas.append(hs)

        in_copy(0, 0).start()
        for i in range(n_blocks):
            slot = i % 2
            if i + 1 < n_blocks:
                in_copy(i + 1, (i + 1) % 2).start()
            in_copy(i, slot).wait()
            xb = xbuf[slot]
            p_col[pl.ds(i * BLK, BLK), :] = jnp.sum(
                xb * xb, axis=1, keepdims=True)
            out_ref[pl.ds(i * BLK, BLK), :] = xb.astype(jnp.bfloat16)
            if (i + 1) % CHUNK_BLKS == 0:
                send_chunk(i // CHUNK_BLKS)

        g = gamma_ref[:, :]
        for c in range(n_chunks):
            csl = pl.ds(c * crows, crows)
            if COMM:
                for rdma in rdmas[c]:
                    rdma.wait()
                total = (p_row[:, csl] + comm_row[0, :, csl] +
                         comm_row[1, :, csl] + comm_row[2, :, csl])
            else:
                total = p_row[:, csl] * 4.0
            inv_row = lax.rsqrt(total * (1.0 / N_GLOBAL) + EPS)
            p_col[csl, :] = lax.dot_general(
                inv_row, e11, (((0,), (0,)), ((), ())),
                preferred_element_type=jnp.float32)
            for i in range(c * CHUNK_BLKS, (c + 1) * CHUNK_BLKS):
                sl = pl.ds(i * BLK, BLK)
                xb = out_ref[sl, :].astype(jnp.float32)
                inv = p_col[sl, :]
                out_ref[sl, :] = (g * xb * inv).astype(jnp.bfloat16)

    return pl.pallas_call(
        body,
        out_shape=jax.ShapeDtypeStruct((m, n_local), jnp.bfloat16),
        in_specs=[
            pl.BlockSpec(memory_space=pl.ANY),
            pl.BlockSpec(memory_space=pltpu.VMEM),
        ],
        out_specs=pl.BlockSpec(memory_space=pltpu.VMEM),
        scratch_shapes=[
            pltpu.VMEM((2, BLK, n_local), jnp.float32),
            pltpu.VMEM((m, 1), jnp.float32),
            pltpu.VMEM((1, m), jnp.float32),
            pltpu.VMEM((N_DEV - 1, 1, m), jnp.float32),
            pltpu.SemaphoreType.DMA((2,)),
            pltpu.SemaphoreType.DMA(
                (N_DEV - 1, m // (CHUNK_BLKS * BLK))),
            pltpu.SemaphoreType.DMA(
                (N_DEV - 1, m // (CHUNK_BLKS * BLK))),
        ],
        compiler_params=pltpu.CompilerParams(
            collective_id=0 if (COMM or BARRIER) else None,
            vmem_limit_bytes=60 * 1024 * 1024,
        ),
    )(x, gamma2d)


# device time: 48320 ns/iter; 1.1415x vs baseline; 1.1050x over previous
import jax
import jax.numpy as jnp
from jax import lax
from jax.experimental import pallas as pl
from jax.experimental.pallas import tpu as pltpu

N_DEV = 4
N_GLOBAL = 8192
EPS = 1e-5
BLK = 512
BARRIER = True
CHUNK_BLKS = 4
COMM = True


def kernel(x, gamma):
    m, n_local = x.shape
    n_blocks = m // BLK
    gamma2d = gamma.reshape(1, n_local)

    def body(x_hbm, gamma_ref, out_hbm, xbuf, xcache, p_col, p_row, comm_row,
             copy_sems, out_sems, send_sems, recv_sems):
        my = lax.axis_index("i")

        if COMM or BARRIER:
            barrier_sem = pltpu.get_barrier_semaphore()
            for o in range(1, N_DEV):
                pl.semaphore_signal(
                    barrier_sem, inc=1,
                    device_id=(lax.rem(my + o, N_DEV),),
                    device_id_type=pl.DeviceIdType.MESH,
                )
            pl.semaphore_wait(barrier_sem, N_DEV - 1)

        def in_copy(i, slot):
            return pltpu.make_async_copy(
                x_hbm.at[pl.ds(i * BLK, BLK), :],
                xbuf.at[slot],
                copy_sems.at[slot],
            )

        e11 = jnp.ones((1, 1), jnp.float32)
        crows = CHUNK_BLKS * BLK
        n_chunks = n_blocks // CHUNK_BLKS
        rdmas = []

        def send_chunk(c):
            csl = pl.ds(c * crows, crows)
            p_row[:, csl] = lax.dot_general(
                e11, p_col[csl, :], (((1,), (1,)), ((), ())),
                preferred_element_type=jnp.float32)
            if not COMM:
                return
            hs = []
            for o in range(1, N_DEV):
                rdma = pltpu.make_async_remote_copy(
                    src_ref=p_row.at[:, csl],
                    dst_ref=comm_row.at[o - 1, :, csl],
                    send_sem=send_sems.at[o - 1, c],
                    recv_sem=recv_sems.at[o - 1, c],
                    device_id=(lax.rem(my + o, N_DEV),),
                    device_id_type=pl.DeviceIdType.MESH,
                )
                rdma.start()
                hs.append(rdma)
            rdmas.append(hs)

        in_copy(0, 0).start()
        for i in range(n_blocks):
            slot = i % 2
            if i + 1 < n_blocks:
                in_copy(i + 1, (i + 1) % 2).start()
            in_copy(i, slot).wait()
            xb = xbuf[slot]
            p_col[pl.ds(i * BLK, BLK), :] = jnp.sum(
                xb * xb, axis=1, keepdims=True)
            xcache[pl.ds(i * BLK, BLK), :] = xb.astype(jnp.bfloat16)
            if (i + 1) % CHUNK_BLKS == 0:
                send_chunk(i // CHUNK_BLKS)

        g = gamma_ref[:, :]
        out_dmas = [None] * n_blocks
        for c in range(n_chunks):
            csl = pl.ds(c * crows, crows)
            if COMM:
                for rdma in rdmas[c]:
                    rdma.wait()
                total = (p_row[:, csl] + comm_row[0, :, csl] +
                         comm_row[1, :, csl] + comm_row[2, :, csl])
            else:
                total = p_row[:, csl] * 4.0
            inv_row = lax.rsqrt(total * (1.0 / N_GLOBAL) + EPS)
            p_col[csl, :] = lax.dot_general(
                inv_row, e11, (((0,), (0,)), ((), ())),
                preferred_element_type=jnp.float32)
            for i in range(c * CHUNK_BLKS, (c + 1) * CHUNK_BLKS):
                sl = pl.ds(i * BLK, BLK)
                xb = xcache[sl, :].astype(jnp.float32)
                inv = p_col[sl, :]
                xcache[sl, :] = (g * xb * inv).astype(jnp.bfloat16)
                if i >= 2:
                    out_dmas[i - 2].wait()
                out_dmas[i] = pltpu.make_async_copy(
                    xcache.at[sl, :], out_hbm.at[sl, :], out_sems.at[i % 2])
                out_dmas[i].start()
        out_dmas[n_blocks - 2].wait()
        out_dmas[n_blocks - 1].wait()

    return pl.pallas_call(
        body,
        out_shape=jax.ShapeDtypeStruct((m, n_local), jnp.bfloat16),
        in_specs=[
            pl.BlockSpec(memory_space=pl.ANY),
            pl.BlockSpec(memory_space=pltpu.VMEM),
        ],
        out_specs=pl.BlockSpec(memory_space=pl.ANY),
        scratch_shapes=[
            pltpu.VMEM((2, BLK, n_local), jnp.float32),
            pltpu.VMEM((m, n_local), jnp.bfloat16),
            pltpu.VMEM((m, 1), jnp.float32),
            pltpu.VMEM((1, m), jnp.float32),
            pltpu.VMEM((N_DEV - 1, 1, m), jnp.float32),
            pltpu.SemaphoreType.DMA((2,)),
            pltpu.SemaphoreType.DMA((2,)),
            pltpu.SemaphoreType.DMA(
                (N_DEV - 1, m // (CHUNK_BLKS * BLK))),
            pltpu.SemaphoreType.DMA(
                (N_DEV - 1, m // (CHUNK_BLKS * BLK))),
        ],
        compiler_params=pltpu.CompilerParams(
            collective_id=0 if (COMM or BARRIER) else None,
            vmem_limit_bytes=60 * 1024 * 1024,
        ),
    )(x, gamma2d)


# device time: 46154 ns/iter; 1.1951x vs baseline; 1.0469x over previous
import jax
import jax.numpy as jnp
from jax import lax
from jax.experimental import pallas as pl
from jax.experimental.pallas import tpu as pltpu

N_DEV = 4
N_GLOBAL = 8192
EPS = 1e-5
BLK = 512
BARRIER = True
CHUNK_BLKS = 4
COMM = True


def kernel(x, gamma):
    m, n_local = x.shape
    n_blocks = m // BLK
    gamma2d = gamma.reshape(1, n_local)

    def body(x_hbm, gamma_ref, out_hbm, xbuf, xcache, p_col, p_row, comm_row,
             copy_sems, out_sems, send_sems, recv_sems):
        my = lax.axis_index("i")

        if COMM or BARRIER:
            barrier_sem = pltpu.get_barrier_semaphore()
            for o in range(1, N_DEV):
                pl.semaphore_signal(
                    barrier_sem, inc=1,
                    device_id=(lax.rem(my + o, N_DEV),),
                    device_id_type=pl.DeviceIdType.MESH,
                )
            pl.semaphore_wait(barrier_sem, N_DEV - 1)

        def in_copy(i, slot):
            return pltpu.make_async_copy(
                x_hbm.at[pl.ds(i * BLK, BLK), :],
                xbuf.at[slot],
                copy_sems.at[slot],
            )

        e11 = jnp.ones((1, 1), jnp.float32)
        crows = CHUNK_BLKS * BLK
        n_chunks = n_blocks // CHUNK_BLKS
        rdmas = []

        def send_chunk(c):
            csl = pl.ds(c * crows, crows)
            p_row[:, csl] = lax.dot_general(
                e11, p_col[csl, :], (((1,), (1,)), ((), ())),
                preferred_element_type=jnp.float32)
            if not COMM:
                return
            hs = []
            for o in range(1, N_DEV):
                rdma = pltpu.make_async_remote_copy(
                    src_ref=p_row.at[:, csl],
                    dst_ref=comm_row.at[o - 1, :, csl],
                    send_sem=send_sems.at[o - 1, c],
                    recv_sem=recv_sems.at[o - 1, c],
                    device_id=(lax.rem(my + o, N_DEV),),
                    device_id_type=pl.DeviceIdType.MESH,
                )
                rdma.start()
                hs.append(rdma)
            rdmas.append(hs)

        g = gamma_ref[:, :]
        in_copy(0, 0).start()
        in_copy(1, 1).start()
        for i in range(n_blocks):
            slot = i % 3
            in_copy(i, slot).wait()
            if i + 2 < n_blocks:
                in_copy(i + 2, (i + 2) % 3).start()
            xb = xbuf[slot]
            p_col[pl.ds(i * BLK, BLK), :] = jnp.sum(
                xb * xb, axis=1, keepdims=True)
            xcache[pl.ds(i * BLK, BLK), :] = (g * xb).astype(jnp.bfloat16)
            if (i + 1) % CHUNK_BLKS == 0:
                send_chunk(i // CHUNK_BLKS)

        out_dmas = [None] * n_blocks
        for c in range(n_chunks):
            csl = pl.ds(c * crows, crows)
            if COMM:
                for rdma in rdmas[c]:
                    rdma.wait()
                total = (p_row[:, csl] + comm_row[0, :, csl] +
                         comm_row[1, :, csl] + comm_row[2, :, csl])
            else:
                total = p_row[:, csl] * 4.0
            inv_row = lax.rsqrt(total * (1.0 / N_GLOBAL) + EPS)
            p_col[csl, :] = lax.dot_general(
                inv_row, e11, (((0,), (0,)), ((), ())),
                preferred_element_type=jnp.float32)
            for i in range(c * CHUNK_BLKS, (c + 1) * CHUNK_BLKS):
                sl = pl.ds(i * BLK, BLK)
                inv_b = p_col[sl, :].astype(jnp.bfloat16)
                xcache[sl, :] = xcache[sl, :] * inv_b
                if i >= 2:
                    out_dmas[i - 2].wait()
                out_dmas[i] = pltpu.make_async_copy(
                    xcache.at[sl, :], out_hbm.at[sl, :], out_sems.at[i % 2])
                out_dmas[i].start()
        out_dmas[n_blocks - 2].wait()
        out_dmas[n_blocks - 1].wait()

    return pl.pallas_call(
        body,
        out_shape=jax.ShapeDtypeStruct((m, n_local), jnp.bfloat16),
        in_specs=[
            pl.BlockSpec(memory_space=pl.ANY),
            pl.BlockSpec(memory_space=pltpu.VMEM),
        ],
        out_specs=pl.BlockSpec(memory_space=pl.ANY),
        scratch_shapes=[
            pltpu.VMEM((3, BLK, n_local), jnp.float32),
            pltpu.VMEM((m, n_local), jnp.bfloat16),
            pltpu.VMEM((m, 1), jnp.float32),
            pltpu.VMEM((1, m), jnp.float32),
            pltpu.VMEM((N_DEV - 1, 1, m), jnp.float32),
            pltpu.SemaphoreType.DMA((3,)),
            pltpu.SemaphoreType.DMA((2,)),
            pltpu.SemaphoreType.DMA(
                (N_DEV - 1, m // (CHUNK_BLKS * BLK))),
            pltpu.SemaphoreType.DMA(
                (N_DEV - 1, m // (CHUNK_BLKS * BLK))),
        ],
        compiler_params=pltpu.CompilerParams(
            collective_id=0 if (COMM or BARRIER) else None,
            vmem_limit_bytes=60 * 1024 * 1024,
        ),
    )(x, gamma2d)
